# baseline (speedup 1.0000x reference)
import jax
import jax.numpy as jnp
from jax import lax
from jax.experimental import pallas as pl

_EPS = 1e-8
_THRESH = 0.9
_BB = 32


def _merge_block_kernel(slots_ref, final_ref, smask_ref):
    BB, S, D = slots_ref.shape
    N = BB * S
    S2 = 2 * S
    NP = N // S2
    X = slots_ref[...].reshape(N, D)
    lane = lax.broadcasted_iota(jnp.int32, (N, S2), 1)
    lane_f = lane.astype(jnp.float32)
    rowid = lax.broadcasted_iota(jnp.int32, (N, 1), 0) & (S2 - 1)
    rowid_f = rowid.astype(jnp.float32)
    valid_f = ((rowid & S) == (lane & S)).astype(jnp.float32)
    inv = lax.rsqrt(jnp.sum(X * X, axis=1, keepdims=True))
    Y = X * inv
    gs = []
    for p in range(NP):
        yp = Y[p * S2:(p + 1) * S2]
        gs.append(lax.dot_general(yp, yp, (((1,), (1,)), ((), ())),
                                  preferred_element_type=jnp.float32))
    G = jnp.concatenate(gs, axis=0)
    maskf = (G > _THRESH).astype(jnp.float32) * valid_f
    count = jnp.sum(maskf, axis=1, keepdims=True)
    multi_f = (count > 1.0).astype(jnp.float32)
    mhat = maskf * (1.0 / (count + _EPS))
    fi = float(S2) - jnp.max(maskf * (S2 - lane_f), axis=1, keepdims=True)
    F = (fi == lane_f).astype(jnp.float32)
    ones_col = jnp.ones((S2, 1), jnp.float32)
    mrs, zcs = [], []
    for p in range(NP):
        sl = slice(p * S2, (p + 1) * S2)
        mb = multi_f[sl]
        mrs.append(lax.dot_general(ones_col, mb, (((1,), (1,)), ((), ())),
                                   preferred_element_type=jnp.float32))
        zcs.append(lax.dot_general(F[sl], mb, (((0,), (0,)), ((), ())),
                                   preferred_element_type=jnp.float32))
    MR = jnp.concatenate(mrs, axis=0)
    ZC = jnp.concatenate(zcs, axis=0)
    wm = maskf * MR
    covered = jnp.sum(wm, axis=1, keepdims=True)
    smask_col = (covered == ZC).astype(jnp.float32)
    slw = jnp.max(wm * (lane_f + 1.0), axis=1, keepdims=True) - 1.0
    s_last = jnp.maximum(slw, rowid_f)
    oh = (s_last == lane_f).astype(jnp.float32)
    for p in range(NP):
        sl = slice(p * S2, (p + 1) * S2)
        w = lax.dot_general(oh[sl], mhat[sl], (((1,), (0,)), ((), ())),
                            preferred_element_type=jnp.float32)
        out = lax.dot_general(w, X[sl], (((1,), (0,)), ((), ())),
                              preferred_element_type=jnp.float32)
        final_ref[2 * p] = out[:S]
        final_ref[2 * p + 1] = out[S:]
    smask_ref[...] = smask_col.reshape(BB, S)


def kernel(slots):
    B, S, D = slots.shape
    grid = (B // _BB,)
    final, smask = pl.pallas_call(
        _merge_block_kernel,
        grid=grid,
        in_specs=[pl.BlockSpec((_BB, S, D), lambda i: (i, 0, 0))],
        out_specs=[
            pl.BlockSpec((_BB, S, D), lambda i: (i, 0, 0)),
            pl.BlockSpec((_BB, S), lambda i: (i, 0)),
        ],
        out_shape=[
            jax.ShapeDtypeStruct((B, S, D), slots.dtype),
            jax.ShapeDtypeStruct((B, S), slots.dtype),
        ],
    )(slots)
    return final, smask


# speedup vs baseline: 1.0400x; 1.0400x over previous
import jax
import jax.numpy as jnp
from jax import lax
from jax.experimental import pallas as pl

_EPS = 1e-8
_THRESH = 0.9
_BB = 128


def _merge_block_kernel(slots_ref, final_ref, smask_ref):
    BB, S, D = slots_ref.shape
    N = BB * S
    S2 = 2 * S
    NP = N // S2
    X = slots_ref[...].reshape(N, D)
    lane = lax.broadcasted_iota(jnp.int32, (N, S2), 1)
    lane_f = lane.astype(jnp.float32)
    rowid = lax.broadcasted_iota(jnp.int32, (N, 1), 0) & (S2 - 1)
    rowid_f = rowid.astype(jnp.float32)
    valid_f = ((rowid & S) == (lane & S)).astype(jnp.float32)
    inv = lax.rsqrt(jnp.sum(X * X, axis=1, keepdims=True))
    Y = X * inv
    gs = []
    for p in range(NP):
        yp = Y[p * S2:(p + 1) * S2]
        gs.append(lax.dot_general(yp, yp, (((1,), (1,)), ((), ())),
                                  preferred_element_type=jnp.float32))
    G = jnp.concatenate(gs, axis=0)
    maskf = (G > _THRESH).astype(jnp.float32) * valid_f
    count = jnp.sum(maskf, axis=1, keepdims=True)
    multi_f = (count > 1.0).astype(jnp.float32)
    mhat = maskf * (1.0 / (count + _EPS))
    fi = float(S2) - jnp.max(maskf * (S2 - lane_f), axis=1, keepdims=True)
    F = (fi == lane_f).astype(jnp.float32)
    ones_col = jnp.ones((S2, 1), jnp.float32)
    mrs, zcs = [], []
    for p in range(NP):
        sl = slice(p * S2, (p + 1) * S2)
        mb = multi_f[sl]
        mrs.append(lax.dot_general(ones_col, mb, (((1,), (1,)), ((), ())),
                                   preferred_element_type=jnp.float32))
        zcs.append(lax.dot_general(F[sl], mb, (((0,), (0,)), ((), ())),
                                   preferred_element_type=jnp.float32))
    MR = jnp.concatenate(mrs, axis=0)
    ZC = jnp.concatenate(zcs, axis=0)
    wm = maskf * MR
    covered = jnp.sum(wm, axis=1, keepdims=True)
    smask_col = (covered == ZC).astype(jnp.float32)
    slw = jnp.max(wm * (lane_f + 1.0), axis=1, keepdims=True) - 1.0
    s_last = jnp.maximum(slw, rowid_f)
    oh = (s_last == lane_f).astype(jnp.float32)
    for p in range(NP):
        sl = slice(p * S2, (p + 1) * S2)
        w = lax.dot_general(oh[sl], mhat[sl], (((1,), (0,)), ((), ())),
                            preferred_element_type=jnp.float32)
        out = lax.dot_general(w, X[sl], (((1,), (0,)), ((), ())),
                              preferred_element_type=jnp.float32)
        final_ref[2 * p] = out[:S]
        final_ref[2 * p + 1] = out[S:]
    smask_ref[...] = smask_col.reshape(BB, S)


def kernel(slots):
    B, S, D = slots.shape
    grid = (B // _BB,)
    final, smask = pl.pallas_call(
        _merge_block_kernel,
        grid=grid,
        in_specs=[pl.BlockSpec((_BB, S, D), lambda i: (i, 0, 0))],
        out_specs=[
            pl.BlockSpec((_BB, S, D), lambda i: (i, 0, 0)),
            pl.BlockSpec((_BB, S), lambda i: (i, 0)),
        ],
        out_shape=[
            jax.ShapeDtypeStruct((B, S, D), slots.dtype),
            jax.ShapeDtypeStruct((B, S), slots.dtype),
        ],
    )(slots)
    return final, smask


# speedup vs baseline: 1.0886x; 1.0468x over previous
import jax
import jax.numpy as jnp
from jax import lax
from jax.experimental import pallas as pl

_EPS = 1e-8
_THRESH = 0.9
_BB = 128


def _merge_block_kernel(slots_ref, final_ref, smask_ref):
    BB, S, D = slots_ref.shape
    N = BB * S
    S2 = 2 * S
    NP = N // S2
    X = slots_ref[...].reshape(N, D)
    lane = lax.broadcasted_iota(jnp.int32, (N, S2), 1)
    lane_f = lane.astype(jnp.float32)
    rowid = lax.broadcasted_iota(jnp.int32, (N, 1), 0) & (S2 - 1)
    rowid_f = rowid.astype(jnp.float32)
    thr = jnp.where((rowid & S) == (lane & S), _THRESH, 3.0)
    inv = lax.rsqrt(jnp.sum(X * X, axis=1, keepdims=True))
    Y = X * inv
    gs = []
    for p in range(NP):
        yp = Y[p * S2:(p + 1) * S2]
        gs.append(lax.dot_general(yp, yp, (((1,), (1,)), ((), ())),
                                  preferred_element_type=jnp.float32))
    G = jnp.concatenate(gs, axis=0)
    maskf = (G > thr).astype(jnp.float32)
    count = jnp.sum(maskf, axis=1, keepdims=True)
    multi_f = (count > 1.0).astype(jnp.float32)
    mhat = maskf * (1.0 / (count + _EPS))
    fi = float(S2) - jnp.max(maskf * (S2 - lane_f), axis=1, keepdims=True)
    F = (fi == lane_f).astype(jnp.float32)
    B = maskf - F
    ones_col = jnp.ones((S2, 1), jnp.float32)
    mb2 = multi_f * (rowid_f + 1.0)
    mrs, zcs = [], []
    for p in range(NP):
        sl = slice(p * S2, (p + 1) * S2)
        mrs.append(lax.dot_general(ones_col, mb2[sl], (((1,), (1,)), ((), ())),
                                   preferred_element_type=jnp.float32))
        zcs.append(lax.dot_general(B[sl], multi_f[sl], (((0,), (0,)), ((), ())),
                                   preferred_element_type=jnp.float32))
    MRL = jnp.concatenate(mrs, axis=0)
    ZC = jnp.concatenate(zcs, axis=0)
    smask_col = (ZC == 0.0).astype(jnp.float32)
    slw = jnp.max(maskf * MRL, axis=1, keepdims=True) - 1.0
    s_last = jnp.maximum(slw, rowid_f)
    oh = (s_last == lane_f).astype(jnp.float32)
    for p in range(NP):
        sl = slice(p * S2, (p + 1) * S2)
        w = lax.dot_general(oh[sl], mhat[sl], (((1,), (0,)), ((), ())),
                            preferred_element_type=jnp.float32)
        out = lax.dot_general(w, X[sl], (((1,), (0,)), ((), ())),
                              preferred_element_type=jnp.float32)
        final_ref[2 * p] = out[:S]
        final_ref[2 * p + 1] = out[S:]
    smask_ref[...] = smask_col.reshape(BB, S)


def kernel(slots):
    B, S, D = slots.shape
    grid = (B // _BB,)
    final, smask = pl.pallas_call(
        _merge_block_kernel,
        grid=grid,
        in_specs=[pl.BlockSpec((_BB, S, D), lambda i: (i, 0, 0))],
        out_specs=[
            pl.BlockSpec((_BB, S, D), lambda i: (i, 0, 0)),
            pl.BlockSpec((_BB, S), lambda i: (i, 0)),
        ],
        out_shape=[
            jax.ShapeDtypeStruct((B, S, D), slots.dtype),
            jax.ShapeDtypeStruct((B, S), slots.dtype),
        ],
    )(slots)
    return final, smask
